# baseline (speedup 1.0000x reference)
import functools

import jax
import jax.numpy as jnp
from jax import lax
from jax.experimental import pallas as pl
from jax.experimental.pallas import tpu as pltpu
from jax.experimental.pallas import tpu_sc as plsc

_B, _N, _K, _CIN, _COUT = 2, 4096, 16, 128, 128
_D = 2 * _COUT
_W = _D // 2
_CP = 3 + _CIN + 1
_EPS = 1e-5

_NC, _NS = 2, 16
_NW = _NC * _NS
_PTS = _B * _N
_PW = _PTS // _NW
_P = 8
_NCHUNK = _PW // _P

_BLKA = 512


def _mm_body(p_ref, w_ref, nbr_ref, ctr_ref):
    r = jnp.dot(p_ref[...], w_ref[...], preferred_element_type=jnp.float32)
    fb = lax.bitcast_convert_type(r[:, :_D], jnp.int32)
    key = fb ^ ((fb >> 31) & jnp.int32(0x7FFFFFFF))
    k16 = (key + jnp.int32(0x8000)) >> 16
    nbr_ref[...] = (k16[:, _W:] << 16) | (k16[:, :_W] & jnp.int32(0xFFFF))
    ctr_ref[...] = r[:, _D:]


def _tables(p_all, w_comb):
    grid = _PTS // _BLKA
    return pl.pallas_call(
        _mm_body,
        grid=(grid,),
        in_specs=[
            pl.BlockSpec((_BLKA, _CP), lambda i: (i, 0)),
            pl.BlockSpec((_CP, 2 * _D), lambda i: (0, 0)),
        ],
        out_specs=[
            pl.BlockSpec((_BLKA, _W), lambda i: (i, 0)),
            pl.BlockSpec((_BLKA, _D), lambda i: (i, 0)),
        ],
        out_shape=[
            jax.ShapeDtypeStruct((_PTS, _W), jnp.int32),
            jax.ShapeDtypeStruct((_PTS, _D), jnp.float32),
        ],
    )(p_all, w_comb)


_NBUF = 4


def _sc_body(nbr_hbm, idx_hbm, out_hbm, idx_all,
             rows0, rows1, rows2, rows3, out0, out1,
             gsem0, gsem1, gsem2, gsem3, osem0, osem1):
    wid = lax.axis_index("s") * _NC + lax.axis_index("c")
    pbase0 = wid * _PW
    rows = (rows0, rows1, rows2, rows3)
    out = (out0, out1)
    gsem = (gsem0, gsem1, gsem2, gsem3)
    osem = (osem0, osem1)

    pltpu.sync_copy(idx_hbm.at[wid], idx_all)

    def fetch(g, buf):
        pltpu.async_copy(nbr_hbm.at[idx_all.at[g]], rows[buf], gsem[buf])

    for g in range(_NBUF - 1):
        fetch(g, g)

    def chunk_body(g, cur, cur_o):
        @pl.when(g + _NBUF - 1 < _NCHUNK)
        def _():
            fetch(g + _NBUF - 1, (cur + _NBUF - 1) % _NBUF)

        pltpu.make_async_copy(nbr_hbm.at[idx_all.at[g]], rows[cur],
                              gsem[cur]).wait()

        @pl.when(g >= 2)
        def _():
            pltpu.make_async_copy(out[cur_o], out_hbm.at[pl.ds(0, _P)],
                                  osem[cur_o]).wait()

        def point_body(p, c2):
            def col_body(c, c3):
                slw = pl.ds(c * 16, 16)
                w0 = rows[cur][p * _K, slw]
                acc_lo = w0 << 16
                acc_hi = w0
                for r in range(1, _K):
                    w = rows[cur][p * _K + r, slw]
                    acc_lo = jnp.maximum(acc_lo, w << 16)
                    acc_hi = jnp.maximum(acc_hi, w)
                out[cur_o][p, slw] = (
                    (acc_hi >> 16) << 16
                ) | lax.shift_right_logical(acc_lo, 16)
                return c3

            return lax.fori_loop(0, _W // 16, col_body, c2)

        lax.fori_loop(0, _P, point_body, 0)
        pltpu.async_copy(out[cur_o], out_hbm.at[pl.ds(pbase0 + g * _P, _P)],
                         osem[cur_o])

    def quad_body(go, carry):
        for b in range(_NBUF):
            chunk_body(_NBUF * go + b, b, b % 2)
        return carry

    lax.fori_loop(0, _NCHUNK // _NBUF, quad_body, 0)

    for b in range(2):
        pltpu.make_async_copy(out[b], out_hbm.at[pl.ds(0, _P)],
                              osem[b]).wait()


def _gather_max(t_nbr, idx_by_worker):
    mesh = plsc.VectorSubcoreMesh(
        core_axis_name="c", subcore_axis_name="s", num_cores=_NC,
        num_subcores=_NS,
    )
    fn = pl.kernel(
        _sc_body,
        out_type=jax.ShapeDtypeStruct((_PTS, _W), jnp.int32),
        mesh=mesh,
        scratch_types=[
            pltpu.VMEM((_NCHUNK, _P * _K), jnp.int32),
            pltpu.VMEM((_P * _K, _W), jnp.int32),
            pltpu.VMEM((_P * _K, _W), jnp.int32),
            pltpu.VMEM((_P * _K, _W), jnp.int32),
            pltpu.VMEM((_P * _K, _W), jnp.int32),
            pltpu.VMEM((_P, _W), jnp.int32),
            pltpu.VMEM((_P, _W), jnp.int32),
            pltpu.SemaphoreType.DMA,
            pltpu.SemaphoreType.DMA,
            pltpu.SemaphoreType.DMA,
            pltpu.SemaphoreType.DMA,
            pltpu.SemaphoreType.DMA,
            pltpu.SemaphoreType.DMA,
        ],
    )
    return fn(t_nbr, idx_by_worker)


def _fin_body(m_ref, ctr_ref, o_ref):
    pk = m_ref[0]
    klo = pk << 16
    khi = (pk >> 16) << 16

    def inv(k):
        fb = (k ^ ((k >> 31) & jnp.int32(0x7FFFFFFF))) & jnp.int32(-65536)
        return lax.bitcast_convert_type(fb, jnp.float32)

    olo = jnp.maximum(inv(klo) + ctr_ref[0, :, :_W], 0.0).T
    ohi = jnp.maximum(inv(khi) + ctr_ref[0, :, _W:], 0.0).T
    for c in range(_W // 16):
        o_ref[0, pl.ds(c * 32, 16), :] = olo[c * 16:(c + 1) * 16, :]
        o_ref[0, pl.ds(c * 32 + 16, 16), :] = ohi[c * 16:(c + 1) * 16, :]


def _finish(m, t_ctr):
    blk = 512
    return pl.pallas_call(
        _fin_body,
        grid=(_B, _N // blk),
        in_specs=[
            pl.BlockSpec((1, blk, _W), lambda b, j: (b, j, 0)),
            pl.BlockSpec((1, blk, _D), lambda b, j: (b, j, 0)),
        ],
        out_specs=pl.BlockSpec((1, _D, blk), lambda b, j: (b, 0, j)),
        out_shape=jax.ShapeDtypeStruct((_B, _D, _N), jnp.float32),
    )(m.reshape(_B, _N, _W), t_ctr.reshape(_B, _N, _D))


@jax.jit
def kernel(fea, x, idx, W1, g1, b1, W2, g2, b2):
    inv = 1.0 / jnp.sqrt(1.0 + _EPS)
    s1 = (g1 * inv)[:, None]
    s2 = (g2 * inv)[:, None]
    w1n = (W1[:, 3:] * s1).T
    w1c = ((W1[:, :3] - W1[:, 3:]) * s1).T
    w2n = (W2[:, _CIN:] * s2).T
    w2c = ((W2[:, :_CIN] - W2[:, _CIN:]) * s2).T

    z31 = jnp.zeros((3, _COUT), jnp.float32)
    z131 = jnp.zeros((_CIN, _COUT), jnp.float32)
    zD = jnp.zeros((_D,), jnp.float32)
    w_nbr = jnp.concatenate(
        [
            jnp.concatenate([w1n, z31], axis=1),
            jnp.concatenate([z131, w2n], axis=1),
            zD[None, :],
        ],
        axis=0,
    )
    w_ctr = jnp.concatenate(
        [
            jnp.concatenate([w1c, z31], axis=1),
            jnp.concatenate([z131, w2c], axis=1),
            jnp.concatenate([b1, b2])[None, :],
        ],
        axis=0,
    )
    j = jnp.arange(_W)
    lo_logical = (j // 16) * 32 + j % 16
    perm = jnp.concatenate([lo_logical, lo_logical + 16])
    w_comb = jnp.concatenate([w_nbr[:, perm], w_ctr[:, perm]], axis=1)

    p_all = jnp.concatenate(
        [
            jnp.swapaxes(x, 1, 2),
            jnp.swapaxes(fea, 1, 2),
            jnp.ones((_B, _N, 1), jnp.float32),
        ],
        axis=2,
    ).reshape(_PTS, _CP)

    t_nbr, t_ctr = _tables(p_all, w_comb)

    idx_by_worker = (
        jnp.swapaxes(idx, 1, 2) + (jnp.arange(_B, dtype=jnp.int32) * _N)[:, None, None]
    ).reshape(_NW, _NCHUNK, _P * _K)

    m = _gather_max(t_nbr, idx_by_worker)
    return _finish(m, t_ctr)


# speedup vs baseline: 1.0385x; 1.0385x over previous
import functools

import jax
import jax.numpy as jnp
from jax import lax
from jax.experimental import pallas as pl
from jax.experimental.pallas import tpu as pltpu
from jax.experimental.pallas import tpu_sc as plsc

_B, _N, _K, _CIN, _COUT = 2, 4096, 16, 128, 128
_D = 2 * _COUT
_W = _D // 2
_CP = 3 + _CIN + 1
_EPS = 1e-5

_NC, _NS = 2, 16
_NW = _NC * _NS
_PTS = _B * _N
_PW = _PTS // _NW
_P = 8
_NCHUNK = _PW // _P

_BLKA = 512


def _mm_body(p_ref, w_ref, nbr_ref, ctr_ref):
    r = jnp.dot(p_ref[...], w_ref[...], preferred_element_type=jnp.float32)
    fb = lax.bitcast_convert_type(r[:, :_D], jnp.int32)
    key = fb ^ ((fb >> 31) & jnp.int32(0x7FFFFFFF))
    k16 = (key + jnp.int32(0x8000)) >> 16
    nbr_ref[...] = (k16[:, _W:] << 16) | (k16[:, :_W] & jnp.int32(0xFFFF))
    ctr_ref[...] = r[:, _D:]


def _tables(p_all, w_comb):
    grid = _PTS // _BLKA
    return pl.pallas_call(
        _mm_body,
        grid=(grid,),
        in_specs=[
            pl.BlockSpec((_BLKA, _CP), lambda i: (i, 0)),
            pl.BlockSpec((_CP, 2 * _D), lambda i: (0, 0)),
        ],
        out_specs=[
            pl.BlockSpec((_BLKA, _W), lambda i: (i, 0)),
            pl.BlockSpec((_BLKA, _D), lambda i: (i, 0)),
        ],
        out_shape=[
            jax.ShapeDtypeStruct((_PTS, _W), jnp.int32),
            jax.ShapeDtypeStruct((_PTS, _D), jnp.float32),
        ],
    )(p_all, w_comb)


_NBUF = 4


def _sc_body(nbr_hbm, idx_hbm, out_hbm, idx_all,
             rows0, rows1, rows2, rows3, out0, out1,
             gsem0, gsem1, gsem2, gsem3, osem0, osem1):
    wid = lax.axis_index("s") * _NC + lax.axis_index("c")
    pbase0 = wid * _PW
    rows = (rows0, rows1, rows2, rows3)
    out = (out0, out1)
    gsem = (gsem0, gsem1, gsem2, gsem3)
    osem = (osem0, osem1)

    pltpu.sync_copy(idx_hbm.at[wid], idx_all)

    def fetch(g, buf):
        pltpu.async_copy(nbr_hbm.at[idx_all.at[g]], rows[buf], gsem[buf])

    for g in range(_NBUF - 1):
        fetch(g, g)

    def chunk_body(g, cur, cur_o):
        @pl.when(g + _NBUF - 1 < _NCHUNK)
        def _():
            fetch(g + _NBUF - 1, (cur + _NBUF - 1) % _NBUF)

        pltpu.make_async_copy(nbr_hbm.at[idx_all.at[g]], rows[cur],
                              gsem[cur]).wait()

        @pl.when(g >= 2)
        def _():
            pltpu.make_async_copy(out[cur_o], out_hbm.at[pl.ds(0, _P)],
                                  osem[cur_o]).wait()

        def tree_max(vs):
            while len(vs) > 1:
                vs = [jnp.maximum(vs[i], vs[i + 1])
                      for i in range(0, len(vs) - 1, 2)] + (
                          [vs[-1]] if len(vs) % 2 else [])
            return vs[0]

        def point_body(p, c2):
            def col_body(c, c3):
                slw = pl.ds(c * 16, 16)
                ws = [rows[cur][p * _K + r, slw] for r in range(_K)]
                acc_lo = tree_max([w << 16 for w in ws])
                acc_hi = tree_max(ws)
                out[cur_o][p, slw] = (
                    (acc_hi >> 16) << 16
                ) | lax.shift_right_logical(acc_lo, 16)
                return c3

            return lax.fori_loop(0, _W // 16, col_body, c2)

        lax.fori_loop(0, _P, point_body, 0)
        pltpu.async_copy(out[cur_o], out_hbm.at[pl.ds(pbase0 + g * _P, _P)],
                         osem[cur_o])

    def quad_body(go, carry):
        for b in range(_NBUF):
            chunk_body(_NBUF * go + b, b, b % 2)
        return carry

    lax.fori_loop(0, _NCHUNK // _NBUF, quad_body, 0)

    for b in range(2):
        pltpu.make_async_copy(out[b], out_hbm.at[pl.ds(0, _P)],
                              osem[b]).wait()


def _gather_max(t_nbr, idx_by_worker):
    mesh = plsc.VectorSubcoreMesh(
        core_axis_name="c", subcore_axis_name="s", num_cores=_NC,
        num_subcores=_NS,
    )
    fn = pl.kernel(
        _sc_body,
        out_type=jax.ShapeDtypeStruct((_PTS, _W), jnp.int32),
        mesh=mesh,
        scratch_types=[
            pltpu.VMEM((_NCHUNK, _P * _K), jnp.int32),
            pltpu.VMEM((_P * _K, _W), jnp.int32),
            pltpu.VMEM((_P * _K, _W), jnp.int32),
            pltpu.VMEM((_P * _K, _W), jnp.int32),
            pltpu.VMEM((_P * _K, _W), jnp.int32),
            pltpu.VMEM((_P, _W), jnp.int32),
            pltpu.VMEM((_P, _W), jnp.int32),
            pltpu.SemaphoreType.DMA,
            pltpu.SemaphoreType.DMA,
            pltpu.SemaphoreType.DMA,
            pltpu.SemaphoreType.DMA,
            pltpu.SemaphoreType.DMA,
            pltpu.SemaphoreType.DMA,
        ],
    )
    return fn(t_nbr, idx_by_worker)


def _fin_body(m_ref, ctr_ref, o_ref):
    pk = m_ref[0]
    klo = pk << 16
    khi = (pk >> 16) << 16

    def inv(k):
        fb = (k ^ ((k >> 31) & jnp.int32(0x7FFFFFFF))) & jnp.int32(-65536)
        return lax.bitcast_convert_type(fb, jnp.float32)

    olo = jnp.maximum(inv(klo) + ctr_ref[0, :, :_W], 0.0).T
    ohi = jnp.maximum(inv(khi) + ctr_ref[0, :, _W:], 0.0).T
    for c in range(_W // 16):
        o_ref[0, pl.ds(c * 32, 16), :] = olo[c * 16:(c + 1) * 16, :]
        o_ref[0, pl.ds(c * 32 + 16, 16), :] = ohi[c * 16:(c + 1) * 16, :]


def _finish(m, t_ctr):
    blk = 512
    return pl.pallas_call(
        _fin_body,
        grid=(_B, _N // blk),
        in_specs=[
            pl.BlockSpec((1, blk, _W), lambda b, j: (b, j, 0)),
            pl.BlockSpec((1, blk, _D), lambda b, j: (b, j, 0)),
        ],
        out_specs=pl.BlockSpec((1, _D, blk), lambda b, j: (b, 0, j)),
        out_shape=jax.ShapeDtypeStruct((_B, _D, _N), jnp.float32),
    )(m.reshape(_B, _N, _W), t_ctr.reshape(_B, _N, _D))


@jax.jit
def kernel(fea, x, idx, W1, g1, b1, W2, g2, b2):
    inv = 1.0 / jnp.sqrt(1.0 + _EPS)
    s1 = (g1 * inv)[:, None]
    s2 = (g2 * inv)[:, None]
    w1n = (W1[:, 3:] * s1).T
    w1c = ((W1[:, :3] - W1[:, 3:]) * s1).T
    w2n = (W2[:, _CIN:] * s2).T
    w2c = ((W2[:, :_CIN] - W2[:, _CIN:]) * s2).T

    z31 = jnp.zeros((3, _COUT), jnp.float32)
    z131 = jnp.zeros((_CIN, _COUT), jnp.float32)
    zD = jnp.zeros((_D,), jnp.float32)
    w_nbr = jnp.concatenate(
        [
            jnp.concatenate([w1n, z31], axis=1),
            jnp.concatenate([z131, w2n], axis=1),
            zD[None, :],
        ],
        axis=0,
    )
    w_ctr = jnp.concatenate(
        [
            jnp.concatenate([w1c, z31], axis=1),
            jnp.concatenate([z131, w2c], axis=1),
            jnp.concatenate([b1, b2])[None, :],
        ],
        axis=0,
    )
    j = jnp.arange(_W)
    lo_logical = (j // 16) * 32 + j % 16
    perm = jnp.concatenate([lo_logical, lo_logical + 16])
    w_comb = jnp.concatenate([w_nbr[:, perm], w_ctr[:, perm]], axis=1)

    p_all = jnp.concatenate(
        [
            jnp.swapaxes(x, 1, 2),
            jnp.swapaxes(fea, 1, 2),
            jnp.ones((_B, _N, 1), jnp.float32),
        ],
        axis=2,
    ).reshape(_PTS, _CP)

    t_nbr, t_ctr = _tables(p_all, w_comb)

    idx_by_worker = (
        jnp.swapaxes(idx, 1, 2) + (jnp.arange(_B, dtype=jnp.int32) * _N)[:, None, None]
    ).reshape(_NW, _NCHUNK, _P * _K)

    m = _gather_max(t_nbr, idx_by_worker)
    return _finish(m, t_ctr)


# speedup vs baseline: 1.1638x; 1.1206x over previous
import functools

import jax
import jax.numpy as jnp
from jax import lax
from jax.experimental import pallas as pl
from jax.experimental.pallas import tpu as pltpu
from jax.experimental.pallas import tpu_sc as plsc

_B, _N, _K, _CIN, _COUT = 2, 4096, 16, 128, 128
_D = 2 * _COUT
_W = _D // 2
_CP = 3 + _CIN + 1
_EPS = 1e-5

_NC, _NS = 2, 16
_NW = _NC * _NS
_PTS = _B * _N
_PW = _PTS // _NW
_P = 8
_NCHUNK = _PW // _P

_BLKA = 512


def _mm_body(p_ref, w_ref, nbr_ref, ctr_ref):
    r = jnp.dot(p_ref[...], w_ref[...], preferred_element_type=jnp.float32)
    fb = lax.bitcast_convert_type(r[:, :_D], jnp.int32)
    key = fb ^ ((fb >> 31) & jnp.int32(0x7FFFFFFF))
    k16 = (key + jnp.int32(0x8000)) >> 16
    nbr_ref[...] = (k16[:, _W:] << 16) | (k16[:, :_W] & jnp.int32(0xFFFF))
    ctr_ref[...] = r[:, _D:]


def _tables(p_all, w_comb):
    grid = _PTS // _BLKA
    return pl.pallas_call(
        _mm_body,
        grid=(grid,),
        in_specs=[
            pl.BlockSpec((_BLKA, _CP), lambda i: (i, 0)),
            pl.BlockSpec((_CP, 2 * _D), lambda i: (0, 0)),
        ],
        out_specs=[
            pl.BlockSpec((_BLKA, _W), lambda i: (i, 0)),
            pl.BlockSpec((_BLKA, _D), lambda i: (i, 0)),
        ],
        out_shape=[
            jax.ShapeDtypeStruct((_PTS, _W), jnp.int32),
            jax.ShapeDtypeStruct((_PTS, _D), jnp.float32),
        ],
    )(p_all, w_comb)


_NBUF = 4


def _sc_body(nbr_hbm, idx_hbm, out_hbm, idx_all,
             rows0, rows1, rows2, rows3, out0, out1,
             gsem0, gsem1, gsem2, gsem3, osem0, osem1):
    wid = lax.axis_index("s") * _NC + lax.axis_index("c")
    pbase0 = wid * _PW
    rows = (rows0, rows1, rows2, rows3)
    out = (out0, out1)
    gsem = (gsem0, gsem1, gsem2, gsem3)
    osem = (osem0, osem1)

    pltpu.sync_copy(idx_hbm.at[wid], idx_all)

    def fetch(g, buf):
        pltpu.async_copy(nbr_hbm.at[idx_all.at[g]], rows[buf], gsem[buf])

    for g in range(_NBUF - 1):
        fetch(g, g)

    def chunk_body(g, cur, cur_o):
        @pl.when(g + _NBUF - 1 < _NCHUNK)
        def _():
            fetch(g + _NBUF - 1, (cur + _NBUF - 1) % _NBUF)

        pltpu.make_async_copy(nbr_hbm.at[idx_all.at[g]], rows[cur],
                              gsem[cur]).wait()

        @pl.when(g >= 2)
        def _():
            pltpu.make_async_copy(out[cur_o], out_hbm.at[pl.ds(0, _P)],
                                  osem[cur_o]).wait()

        def tree_max(vs):
            while len(vs) > 1:
                vs = [jnp.maximum(vs[i], vs[i + 1])
                      for i in range(0, len(vs) - 1, 2)] + (
                          [vs[-1]] if len(vs) % 2 else [])
            return vs[0]

        @plsc.parallel_loop(0, _P * (_W // 16), unroll=2)
        def _(i):
            p = i // (_W // 16)
            c = i % (_W // 16)
            slw = pl.ds(c * 16, 16)
            ws = [rows[cur][p * _K + r, slw] for r in range(_K)]
            acc_lo = tree_max([w << 16 for w in ws])
            acc_hi = tree_max(ws)
            out[cur_o][p, slw] = (
                (acc_hi >> 16) << 16
            ) | lax.shift_right_logical(acc_lo, 16)
        pltpu.async_copy(out[cur_o], out_hbm.at[pl.ds(pbase0 + g * _P, _P)],
                         osem[cur_o])

    def quad_body(go, carry):
        for b in range(_NBUF):
            chunk_body(_NBUF * go + b, b, b % 2)
        return carry

    lax.fori_loop(0, _NCHUNK // _NBUF, quad_body, 0)

    for b in range(2):
        pltpu.make_async_copy(out[b], out_hbm.at[pl.ds(0, _P)],
                              osem[b]).wait()


def _gather_max(t_nbr, idx_by_worker):
    mesh = plsc.VectorSubcoreMesh(
        core_axis_name="c", subcore_axis_name="s", num_cores=_NC,
        num_subcores=_NS,
    )
    fn = pl.kernel(
        _sc_body,
        out_type=jax.ShapeDtypeStruct((_PTS, _W), jnp.int32),
        mesh=mesh,
        scratch_types=[
            pltpu.VMEM((_NCHUNK, _P * _K), jnp.int32),
            pltpu.VMEM((_P * _K, _W), jnp.int32),
            pltpu.VMEM((_P * _K, _W), jnp.int32),
            pltpu.VMEM((_P * _K, _W), jnp.int32),
            pltpu.VMEM((_P * _K, _W), jnp.int32),
            pltpu.VMEM((_P, _W), jnp.int32),
            pltpu.VMEM((_P, _W), jnp.int32),
            pltpu.SemaphoreType.DMA,
            pltpu.SemaphoreType.DMA,
            pltpu.SemaphoreType.DMA,
            pltpu.SemaphoreType.DMA,
            pltpu.SemaphoreType.DMA,
            pltpu.SemaphoreType.DMA,
        ],
    )
    return fn(t_nbr, idx_by_worker)


def _fin_body(m_ref, ctr_ref, o_ref):
    pk = m_ref[0]
    klo = pk << 16
    khi = (pk >> 16) << 16

    def inv(k):
        fb = (k ^ ((k >> 31) & jnp.int32(0x7FFFFFFF))) & jnp.int32(-65536)
        return lax.bitcast_convert_type(fb, jnp.float32)

    olo = jnp.maximum(inv(klo) + ctr_ref[0, :, :_W], 0.0).T
    ohi = jnp.maximum(inv(khi) + ctr_ref[0, :, _W:], 0.0).T
    for c in range(_W // 16):
        o_ref[0, pl.ds(c * 32, 16), :] = olo[c * 16:(c + 1) * 16, :]
        o_ref[0, pl.ds(c * 32 + 16, 16), :] = ohi[c * 16:(c + 1) * 16, :]


def _finish(m, t_ctr):
    blk = 512
    return pl.pallas_call(
        _fin_body,
        grid=(_B, _N // blk),
        in_specs=[
            pl.BlockSpec((1, blk, _W), lambda b, j: (b, j, 0)),
            pl.BlockSpec((1, blk, _D), lambda b, j: (b, j, 0)),
        ],
        out_specs=pl.BlockSpec((1, _D, blk), lambda b, j: (b, 0, j)),
        out_shape=jax.ShapeDtypeStruct((_B, _D, _N), jnp.float32),
    )(m.reshape(_B, _N, _W), t_ctr.reshape(_B, _N, _D))


@jax.jit
def kernel(fea, x, idx, W1, g1, b1, W2, g2, b2):
    inv = 1.0 / jnp.sqrt(1.0 + _EPS)
    s1 = (g1 * inv)[:, None]
    s2 = (g2 * inv)[:, None]
    w1n = (W1[:, 3:] * s1).T
    w1c = ((W1[:, :3] - W1[:, 3:]) * s1).T
    w2n = (W2[:, _CIN:] * s2).T
    w2c = ((W2[:, :_CIN] - W2[:, _CIN:]) * s2).T

    z31 = jnp.zeros((3, _COUT), jnp.float32)
    z131 = jnp.zeros((_CIN, _COUT), jnp.float32)
    zD = jnp.zeros((_D,), jnp.float32)
    w_nbr = jnp.concatenate(
        [
            jnp.concatenate([w1n, z31], axis=1),
            jnp.concatenate([z131, w2n], axis=1),
            zD[None, :],
        ],
        axis=0,
    )
    w_ctr = jnp.concatenate(
        [
            jnp.concatenate([w1c, z31], axis=1),
            jnp.concatenate([z131, w2c], axis=1),
            jnp.concatenate([b1, b2])[None, :],
        ],
        axis=0,
    )
    j = jnp.arange(_W)
    lo_logical = (j // 16) * 32 + j % 16
    perm = jnp.concatenate([lo_logical, lo_logical + 16])
    w_comb = jnp.concatenate([w_nbr[:, perm], w_ctr[:, perm]], axis=1)

    p_all = jnp.concatenate(
        [
            jnp.swapaxes(x, 1, 2),
            jnp.swapaxes(fea, 1, 2),
            jnp.ones((_B, _N, 1), jnp.float32),
        ],
        axis=2,
    ).reshape(_PTS, _CP)

    t_nbr, t_ctr = _tables(p_all, w_comb)

    idx_by_worker = (
        jnp.swapaxes(idx, 1, 2) + (jnp.arange(_B, dtype=jnp.int32) * _N)[:, None, None]
    ).reshape(_NW, _NCHUNK, _P * _K)

    m = _gather_max(t_nbr, idx_by_worker)
    return _finish(m, t_ctr)
